# baseline (speedup 1.0000x reference)
import functools

import jax
import jax.numpy as jnp
from jax import lax
from jax.experimental import pallas as pl
from jax.experimental.pallas import tpu as pltpu
from jax.experimental.pallas import tpu_sc as plsc

_N = 10000
_D = 128
_E = 160000
_A = 4
_C = 32
_NCHUNK = _E // _C
_NW = 32
_RPT = 624


def _pre_body(x_ref, pos_ref, w2_ref, b2_ref, alpha_ref, ts_ref, pre2_ref, *, bn):
    i = pl.program_id(0)
    alpha = alpha_ref[0, 0]
    rows = lax.broadcasted_iota(jnp.int32, (bn, 1), 0) + i * bn
    x = jnp.where(rows == 0, 0.0, x_ref[...])
    for a in range(_A):
        ts = alpha * x + (1.0 - alpha) * pos_ref[a + 1][None]
        ts_ref[a] = jnp.where(rows == 0, 1.0 - alpha, ts)
    pre2_ref[...] = (
        jnp.dot(x, w2_ref[...], preferred_element_type=jnp.float32,
                precision=lax.Precision.HIGHEST)
        + b2_ref[...]
    )


def _prestage(x, pos_pad, w2, b2, alpha_arr):
    bn = 1000
    nb = _N // bn
    return pl.pallas_call(
        functools.partial(_pre_body, bn=bn),
        grid=(nb,),
        in_specs=[
            pl.BlockSpec((bn, _D), lambda i: (i, 0)),
            pl.BlockSpec((8, _D), lambda i: (0, 0)),
            pl.BlockSpec((_D, _D), lambda i: (0, 0)),
            pl.BlockSpec((1, _D), lambda i: (0, 0)),
            pl.BlockSpec(memory_space=pltpu.SMEM),
        ],
        out_specs=[
            pl.BlockSpec((_A, bn, _D), lambda i: (0, i, 0)),
            pl.BlockSpec((bn, _D), lambda i: (i, 0)),
        ],
        out_shape=[
            jax.ShapeDtypeStruct((_A, _N, _D), jnp.float32),
            jax.ShapeDtypeStruct((_N, _D), jnp.float32),
        ],
    )(x, pos_pad, w2, b2, alpha_arr)


_S = 6
_NSC = 26
_PER_W = _S * _NSC
_TAIL = _NCHUNK - _PER_W * _NW


def _compute_chunk(g_v, relt_v, iv, j):
    def _edge(e, _):
        rel_e = iv[j, 2, pl.ds(e, 16)][0]
        r0 = 4 * e
        for jj in range(8):
            sl = pl.ds(16 * jj, 16)
            s0 = g_v[r0, sl]
            s1 = g_v[r0 + 1, sl]
            s2 = g_v[r0 + 2, sl]
            s3 = g_v[r0 + 3, sl]
            rr = relt_v[rel_e, sl]
            p01r = (s0 * s1) * rr
            p23r = (s2 * s3) * rr
            g_v[r0, sl] = s1 * p23r
            g_v[r0 + 1, sl] = s0 * p23r
            g_v[r0 + 2, sl] = p01r * s3
            g_v[r0 + 3, sl] = p01r * s2
        return 0
    lax.fori_loop(0, _C, _edge, 0)


def _sc_body(ts_hbm, relt_hbm, idx3_hbm, out_hbm,
             g_v0, g_v1, relt_v, i_v0, i_v1, agg_sh,
             sem_ts0, sem_ts1, sem_idx0, sem_idx1,
             sem_sc0, sem_sc1):
    c = lax.axis_index("c")
    s = lax.axis_index("s")
    wid = s * 2 + c
    g_v = (g_v0, g_v1)
    i_v = (i_v0, i_v1)
    sem_ts = (sem_ts0, sem_ts1)
    sem_idx = (sem_idx0, sem_idx1)
    sem_sc = (sem_sc0, sem_sc1)
    pltpu.sync_copy(relt_hbm, relt_v)

    def _zrow(i, _):
        for j in range(8):
            g_v0[i, pl.ds(16 * j, 16)] = jnp.zeros((16,), jnp.float32)
        return 0
    lax.fori_loop(0, 16, _zrow, 0)
    zbase = s * _RPT
    nblk = jnp.where(s == 15, 40, 39)

    def _zcopy(i, _):
        pltpu.sync_copy(g_v0.at[pl.ds(0, 16)],
                        agg_sh.at[pl.ds(zbase + i * 16, 16)])
        return 0
    lax.fori_loop(0, nblk, _zcopy, 0)
    plsc.subcore_barrier()

    start_sc = wid * _NSC

    def _fire_gather(b, ip, j):
        pltpu.async_copy(ts_hbm.at[i_v[ip].at[j, 0]], g_v[b], sem_ts[b])

    def _wait_gather(b):
        pltpu.make_async_copy(ts_hbm.at[i_v[0].at[0, 0]], g_v[b],
                              sem_ts[b]).wait()

    def _fire_scatter(b, ip, j):
        pltpu.async_copy(g_v[b], agg_sh.at[i_v[ip].at[j, 1]], sem_sc[b],
                         add=True)

    def _wait_scatter(b):
        pltpu.make_async_copy(g_v[b], agg_sh.at[i_v[0].at[0, 1]],
                              sem_sc[b]).wait()

    pltpu.sync_copy(idx3_hbm.at[pl.ds(start_sc * _S, _S)], i_v[0])
    _fire_gather(0, 0, 0)

    def _superpair(t, _):
        for p in range(2):
            sc = 2 * t + p
            ip = p
            def _pf():
                pltpu.async_copy(
                    idx3_hbm.at[pl.ds((start_sc + sc + 1) * _S, _S)],
                    i_v[1 - ip], sem_idx[1 - ip])
            if p == 0:
                _pf()
            else:
                pl.when(t < _NSC // 2 - 1)(_pf)
            for j in range(_S):
                b = j % 2
                _wait_gather(b)
                def _wprev():
                    _wait_scatter(1 - b)
                if p == 0 and j == 0:
                    pl.when(t > 0)(_wprev)
                else:
                    _wprev()
                if j < _S - 1:
                    _fire_gather(1 - b, ip, j + 1)
                else:
                    def _xfire():
                        pltpu.make_async_copy(
                            idx3_hbm.at[pl.ds(0, _S)], i_v[1 - ip],
                            sem_idx[1 - ip]).wait()
                        _fire_gather(1 - b, 1 - ip, 0)
                    if p == 0:
                        _xfire()
                    else:
                        pl.when(t < _NSC // 2 - 1)(_xfire)
                _compute_chunk(g_v[b], relt_v, i_v[ip], j)
                _fire_scatter(b, ip, j)
        return 0
    lax.fori_loop(0, _NSC // 2, _superpair, 0, unroll=False)
    _wait_scatter(1)

    @pl.when(wid < _TAIL)
    def _tail_chunk():
        tc = _PER_W * _NW + wid
        pltpu.sync_copy(idx3_hbm.at[pl.ds(tc, 1)], i_v[0].at[pl.ds(0, 1)])
        pltpu.async_copy(ts_hbm.at[i_v[0].at[0, 0]], g_v[0],
                         sem_ts[0]).wait()
        _compute_chunk(g_v[0], relt_v, i_v[0], 0)
        pltpu.sync_copy(g_v[0], agg_sh.at[i_v[0].at[0, 1]], add=True)

    plsc.subcore_barrier()
    rbase = s * _RPT
    pltpu.sync_copy(agg_sh.at[pl.ds(rbase, _RPT)],
                    out_hbm.at[pl.ds(c * _N + rbase, _RPT)])

    @pl.when(s == 15)
    def _tail():
        pltpu.sync_copy(agg_sh.at[pl.ds(16 * _RPT, _N - 16 * _RPT)],
                        out_hbm.at[pl.ds(c * _N + 16 * _RPT, _N - 16 * _RPT)])


def _sc_stage(ts, relt, idx3):
    mesh = plsc.VectorSubcoreMesh(core_axis_name="c", subcore_axis_name="s")
    f = pl.kernel(
        _sc_body,
        out_type=jax.ShapeDtypeStruct((2 * _N, _D), jnp.float32),
        mesh=mesh,
        scratch_types=[
            pltpu.VMEM((4 * _C, _D), jnp.float32),
            pltpu.VMEM((4 * _C, _D), jnp.float32),
            pltpu.VMEM((32, _D), jnp.float32),
            pltpu.VMEM((_S, 3, 4 * _C), jnp.int32),
            pltpu.VMEM((_S, 3, 4 * _C), jnp.int32),
            pltpu.VMEM_SHARED((_N, _D), jnp.float32),
            pltpu.SemaphoreType.DMA,
            pltpu.SemaphoreType.DMA,
            pltpu.SemaphoreType.DMA,
            pltpu.SemaphoreType.DMA,
            pltpu.SemaphoreType.DMA,
            pltpu.SemaphoreType.DMA,
        ],
    )
    return f(ts, relt, idx3)


def _post_body(p0_ref, p1_ref, pre2_ref, w1_ref, g_ref, bt_ref, out_ref, *, bn):
    i = pl.program_id(0)
    rows = lax.broadcasted_iota(jnp.int32, (bn, 1), 0) + i * bn
    agg = p0_ref[...] + p1_ref[...]
    agg = jnp.where(rows == 0, 0.0, agg)
    h = (
        jnp.dot(agg, w1_ref[...], preferred_element_type=jnp.float32,
                precision=lax.Precision.HIGHEST)
        + pre2_ref[...]
    )
    mu = jnp.mean(h, axis=-1, keepdims=True)
    dlt = h - mu
    var = jnp.mean(dlt * dlt, axis=-1, keepdims=True)
    out_ref[...] = dlt * lax.rsqrt(var + 1e-5) * g_ref[...] + bt_ref[...]


def _poststage(partials, pre2, w1, gamma2, beta2):
    bn = 1000
    nb = _N // bn
    return pl.pallas_call(
        functools.partial(_post_body, bn=bn),
        grid=(nb,),
        in_specs=[
            pl.BlockSpec((bn, _D), lambda i: (i, 0)),
            pl.BlockSpec((bn, _D), lambda i: (nb + i, 0)),
            pl.BlockSpec((bn, _D), lambda i: (i, 0)),
            pl.BlockSpec((_D, _D), lambda i: (0, 0)),
            pl.BlockSpec((1, _D), lambda i: (0, 0)),
            pl.BlockSpec((1, _D), lambda i: (0, 0)),
        ],
        out_specs=pl.BlockSpec((bn, _D), lambda i: (i, 0)),
        out_shape=jax.ShapeDtypeStruct((_N, _D), jnp.float32),
    )(partials, partials, pre2, w1, gamma2, beta2)


def kernel(node_features, query, edge_list, rel, rel_emb, pos_emb, alpha,
           W, b, gamma, beta):
    B, N, d = node_features.shape
    E, A = edge_list.shape
    x = node_features.reshape(N, d)
    pos_pad = jnp.zeros((8, d), jnp.float32).at[: A + 1].set(pos_emb)
    relt = rel_emb.at[0].set(jnp.ones((d,), rel_emb.dtype))
    w1 = W[:d]
    w2 = W[d:]
    alpha_arr = jnp.asarray(alpha, jnp.float32).reshape(1, 1)
    b2 = b.reshape(1, d)
    el = edge_list.astype(jnp.int32)
    gidx = (el + (jnp.arange(A, dtype=jnp.int32) * N)[None, :]).reshape(
        _NCHUNK, 1, 4 * _C)
    sidx = el.reshape(_NCHUNK, 1, 4 * _C)
    relp = jnp.pad(rel.astype(jnp.int32).reshape(_NCHUNK, _C),
                   ((0, 0), (0, 3 * _C))).reshape(_NCHUNK, 1, 4 * _C)
    idx3 = jnp.concatenate([gidx, sidx, relp], axis=1)

    ts3, pre2 = _prestage(x, pos_pad, w2, b2, alpha_arr)
    ts = ts3.reshape(A * N, d)
    idx3 = pltpu.with_memory_space_constraint(idx3, pltpu.HBM)
    partials = _sc_stage(ts, relt, idx3)
    out = _poststage(partials, pre2, w1, gamma.reshape(1, d),
                     beta.reshape(1, d))
    return out.reshape(B, N, d)


# speedup vs baseline: 1.0878x; 1.0878x over previous
import functools

import jax
import jax.numpy as jnp
from jax import lax
from jax.experimental import pallas as pl
from jax.experimental.pallas import tpu as pltpu
from jax.experimental.pallas import tpu_sc as plsc

_N = 10000
_D = 128
_E = 160000
_A = 4
_C = 32
_NCHUNK = _E // _C
_NW = 32
_RPT = 624


def _pre_body(x_ref, pos_ref, w2_ref, b2_ref, alpha_ref, ts_ref, pre2_ref, *, bn):
    i = pl.program_id(0)
    alpha = alpha_ref[0, 0]
    rows = lax.broadcasted_iota(jnp.int32, (bn, 1), 0) + i * bn
    x = jnp.where(rows == 0, 0.0, x_ref[...])
    for a in range(_A):
        ts = alpha * x + (1.0 - alpha) * pos_ref[a + 1][None]
        ts_ref[a] = jnp.where(rows == 0, 1.0 - alpha, ts)
    pre2_ref[...] = (
        jnp.dot(x, w2_ref[...], preferred_element_type=jnp.float32,
                precision=lax.Precision.HIGHEST)
        + b2_ref[...]
    )


def _prestage(x, pos_pad, w2, b2, alpha_arr):
    bn = 1000
    nb = _N // bn
    return pl.pallas_call(
        functools.partial(_pre_body, bn=bn),
        grid=(nb,),
        in_specs=[
            pl.BlockSpec((bn, _D), lambda i: (i, 0)),
            pl.BlockSpec((8, _D), lambda i: (0, 0)),
            pl.BlockSpec((_D, _D), lambda i: (0, 0)),
            pl.BlockSpec((1, _D), lambda i: (0, 0)),
            pl.BlockSpec(memory_space=pltpu.SMEM),
        ],
        out_specs=[
            pl.BlockSpec((_A, bn, _D), lambda i: (0, i, 0)),
            pl.BlockSpec((bn, _D), lambda i: (i, 0)),
        ],
        out_shape=[
            jax.ShapeDtypeStruct((_A, _N, _D), jnp.float32),
            jax.ShapeDtypeStruct((_N, _D), jnp.float32),
        ],
    )(x, pos_pad, w2, b2, alpha_arr)


_S = 6
_NSC = 26
_PER_W = _S * _NSC
_TAIL = _NCHUNK - _PER_W * _NW


def _compute_chunk(g_v, r_v):
    def _edge(e, _):
        r0 = 4 * e
        for jj in range(8):
            sl = pl.ds(16 * jj, 16)
            s0 = g_v[r0, sl]
            s1 = g_v[r0 + 1, sl]
            s2 = g_v[r0 + 2, sl]
            s3 = g_v[r0 + 3, sl]
            rr = r_v[e, sl]
            p01r = (s0 * s1) * rr
            p23r = (s2 * s3) * rr
            g_v[r0, sl] = s1 * p23r
            g_v[r0 + 1, sl] = s0 * p23r
            g_v[r0 + 2, sl] = p01r * s3
            g_v[r0 + 3, sl] = p01r * s2
        return 0
    lax.fori_loop(0, _C, _edge, 0)


def _sc_body(ts_hbm, relt_hbm, idx3_hbm, out_hbm,
             g_v0, g_v1, r_v0, r_v1, i_v0, i_v1, relt_sh, agg_sh,
             sem_ts0, sem_ts1, sem_rel0, sem_rel1, sem_idx0, sem_idx1,
             sem_sc0, sem_sc1):
    c = lax.axis_index("c")
    s = lax.axis_index("s")
    wid = s * 2 + c
    g_v = (g_v0, g_v1)
    r_v = (r_v0, r_v1)
    i_v = (i_v0, i_v1)
    sem_ts = (sem_ts0, sem_ts1)
    sem_rel = (sem_rel0, sem_rel1)
    sem_idx = (sem_idx0, sem_idx1)
    sem_sc = (sem_sc0, sem_sc1)
    @pl.when(s == 0)
    def _stage_relt():
        pltpu.sync_copy(relt_hbm, relt_sh)

    def _zrow(i, _):
        for j in range(8):
            g_v0[i, pl.ds(16 * j, 16)] = jnp.zeros((16,), jnp.float32)
        return 0
    lax.fori_loop(0, 16, _zrow, 0)
    zbase = s * _RPT
    nblk = jnp.where(s == 15, 40, 39)

    def _zcopy(i, _):
        pltpu.sync_copy(g_v0.at[pl.ds(0, 16)],
                        agg_sh.at[pl.ds(zbase + i * 16, 16)])
        return 0
    lax.fori_loop(0, nblk, _zcopy, 0)
    plsc.subcore_barrier()

    start_sc = wid * _NSC

    def _fire_gather(b, ip, j):
        pltpu.async_copy(ts_hbm.at[i_v[ip].at[j, 0]], g_v[b], sem_ts[b])
        pltpu.async_copy(relt_sh.at[i_v[ip].at[j, 2, pl.ds(0, 32)]],
                         r_v[b], sem_rel[b])

    def _wait_gather(b):
        pltpu.make_async_copy(ts_hbm.at[i_v[0].at[0, 0]], g_v[b],
                              sem_ts[b]).wait()
        pltpu.make_async_copy(relt_sh.at[i_v[0].at[0, 2, pl.ds(0, 32)]],
                              r_v[b], sem_rel[b]).wait()

    def _fire_scatter(b, ip, j):
        pltpu.async_copy(g_v[b], agg_sh.at[i_v[ip].at[j, 1]], sem_sc[b],
                         add=True)

    def _wait_scatter(b):
        pltpu.make_async_copy(g_v[b], agg_sh.at[i_v[0].at[0, 1]],
                              sem_sc[b]).wait()

    pltpu.sync_copy(idx3_hbm.at[pl.ds(start_sc * _S, _S)], i_v[0])
    _fire_gather(0, 0, 0)

    def _superpair(t, _):
        for p in range(2):
            sc = 2 * t + p
            ip = p
            def _pf():
                pltpu.async_copy(
                    idx3_hbm.at[pl.ds((start_sc + sc + 1) * _S, _S)],
                    i_v[1 - ip], sem_idx[1 - ip])
            if p == 0:
                _pf()
            else:
                pl.when(t < _NSC // 2 - 1)(_pf)
            for j in range(_S):
                b = j % 2
                _wait_gather(b)
                def _wprev():
                    _wait_scatter(1 - b)
                if p == 0 and j == 0:
                    pl.when(t > 0)(_wprev)
                else:
                    _wprev()
                if j < _S - 1:
                    _fire_gather(1 - b, ip, j + 1)
                else:
                    def _xfire():
                        pltpu.make_async_copy(
                            idx3_hbm.at[pl.ds(0, _S)], i_v[1 - ip],
                            sem_idx[1 - ip]).wait()
                        _fire_gather(1 - b, 1 - ip, 0)
                    if p == 0:
                        _xfire()
                    else:
                        pl.when(t < _NSC // 2 - 1)(_xfire)
                _compute_chunk(g_v[b], r_v[b])
                _fire_scatter(b, ip, j)
        return 0
    lax.fori_loop(0, _NSC // 2, _superpair, 0, unroll=False)
    _wait_scatter(1)

    @pl.when(wid < _TAIL)
    def _tail_chunk():
        tc = _PER_W * _NW + wid
        pltpu.sync_copy(idx3_hbm.at[pl.ds(tc, 1)], i_v[0].at[pl.ds(0, 1)])
        ts_cp = pltpu.async_copy(ts_hbm.at[i_v[0].at[0, 0]], g_v[0],
                                 sem_ts[0])
        rel_cp = pltpu.async_copy(relt_sh.at[i_v[0].at[0, 2, pl.ds(0, 32)]],
                                  r_v[0], sem_rel[0])
        ts_cp.wait()
        rel_cp.wait()
        _compute_chunk(g_v[0], r_v[0])
        pltpu.sync_copy(g_v[0], agg_sh.at[i_v[0].at[0, 1]], add=True)

    plsc.subcore_barrier()
    rbase = s * _RPT
    pltpu.sync_copy(agg_sh.at[pl.ds(rbase, _RPT)],
                    out_hbm.at[pl.ds(c * _N + rbase, _RPT)])

    @pl.when(s == 15)
    def _tail():
        pltpu.sync_copy(agg_sh.at[pl.ds(16 * _RPT, _N - 16 * _RPT)],
                        out_hbm.at[pl.ds(c * _N + 16 * _RPT, _N - 16 * _RPT)])


def _sc_stage(ts, relt, idx3):
    mesh = plsc.VectorSubcoreMesh(core_axis_name="c", subcore_axis_name="s")
    f = pl.kernel(
        _sc_body,
        out_type=jax.ShapeDtypeStruct((2 * _N, _D), jnp.float32),
        mesh=mesh,
        scratch_types=[
            pltpu.VMEM((4 * _C, _D), jnp.float32),
            pltpu.VMEM((4 * _C, _D), jnp.float32),
            pltpu.VMEM((_C, _D), jnp.float32),
            pltpu.VMEM((_C, _D), jnp.float32),
            pltpu.VMEM((_S, 3, 4 * _C), jnp.int32),
            pltpu.VMEM((_S, 3, 4 * _C), jnp.int32),
            pltpu.VMEM_SHARED((32, _D), jnp.float32),
            pltpu.VMEM_SHARED((_N, _D), jnp.float32),
            pltpu.SemaphoreType.DMA,
            pltpu.SemaphoreType.DMA,
            pltpu.SemaphoreType.DMA,
            pltpu.SemaphoreType.DMA,
            pltpu.SemaphoreType.DMA,
            pltpu.SemaphoreType.DMA,
            pltpu.SemaphoreType.DMA,
            pltpu.SemaphoreType.DMA,
        ],
    )
    return f(ts, relt, idx3)


def _post_body(p0_ref, p1_ref, pre2_ref, w1_ref, g_ref, bt_ref, out_ref, *, bn):
    i = pl.program_id(0)
    rows = lax.broadcasted_iota(jnp.int32, (bn, 1), 0) + i * bn
    agg = p0_ref[...] + p1_ref[...]
    agg = jnp.where(rows == 0, 0.0, agg)
    h = (
        jnp.dot(agg, w1_ref[...], preferred_element_type=jnp.float32,
                precision=lax.Precision.HIGHEST)
        + pre2_ref[...]
    )
    mu = jnp.mean(h, axis=-1, keepdims=True)
    dlt = h - mu
    var = jnp.mean(dlt * dlt, axis=-1, keepdims=True)
    out_ref[...] = dlt * lax.rsqrt(var + 1e-5) * g_ref[...] + bt_ref[...]


def _poststage(partials, pre2, w1, gamma2, beta2):
    bn = 1000
    nb = _N // bn
    return pl.pallas_call(
        functools.partial(_post_body, bn=bn),
        grid=(nb,),
        in_specs=[
            pl.BlockSpec((bn, _D), lambda i: (i, 0)),
            pl.BlockSpec((bn, _D), lambda i: (nb + i, 0)),
            pl.BlockSpec((bn, _D), lambda i: (i, 0)),
            pl.BlockSpec((_D, _D), lambda i: (0, 0)),
            pl.BlockSpec((1, _D), lambda i: (0, 0)),
            pl.BlockSpec((1, _D), lambda i: (0, 0)),
        ],
        out_specs=pl.BlockSpec((bn, _D), lambda i: (i, 0)),
        out_shape=jax.ShapeDtypeStruct((_N, _D), jnp.float32),
    )(partials, partials, pre2, w1, gamma2, beta2)


def kernel(node_features, query, edge_list, rel, rel_emb, pos_emb, alpha,
           W, b, gamma, beta):
    B, N, d = node_features.shape
    E, A = edge_list.shape
    x = node_features.reshape(N, d)
    pos_pad = jnp.zeros((8, d), jnp.float32).at[: A + 1].set(pos_emb)
    relt = rel_emb.at[0].set(jnp.ones((d,), rel_emb.dtype))
    w1 = W[:d]
    w2 = W[d:]
    alpha_arr = jnp.asarray(alpha, jnp.float32).reshape(1, 1)
    b2 = b.reshape(1, d)
    el = edge_list.astype(jnp.int32)
    gidx = (el + (jnp.arange(A, dtype=jnp.int32) * N)[None, :]).reshape(
        _NCHUNK, 1, 4 * _C)
    sidx = el.reshape(_NCHUNK, 1, 4 * _C)
    relp = jnp.pad(rel.astype(jnp.int32).reshape(_NCHUNK, _C),
                   ((0, 0), (0, 3 * _C))).reshape(_NCHUNK, 1, 4 * _C)
    idx3 = jnp.concatenate([gidx, sidx, relp], axis=1)

    ts3, pre2 = _prestage(x, pos_pad, w2, b2, alpha_arr)
    ts = ts3.reshape(A * N, d)
    idx3 = pltpu.with_memory_space_constraint(idx3, pltpu.HBM)
    partials = _sc_stage(ts, relt, idx3)
    out = _poststage(partials, pre2, w1, gamma.reshape(1, d),
                     beta.reshape(1, d))
    return out.reshape(B, N, d)


# speedup vs baseline: 1.2778x; 1.1747x over previous
import functools

import jax
import jax.numpy as jnp
from jax import lax
from jax.experimental import pallas as pl
from jax.experimental.pallas import tpu as pltpu
from jax.experimental.pallas import tpu_sc as plsc

_N = 10000
_D = 128
_E = 160000
_A = 4
_C = 32
_NCHUNK = _E // _C
_NW = 32
_RPT = 624


def _pre_body(x_ref, pos_ref, w2_ref, b2_ref, alpha_ref, ts_ref, pre2_ref, *, bn):
    i = pl.program_id(0)
    alpha = alpha_ref[0, 0]
    rows = lax.broadcasted_iota(jnp.int32, (bn, 1), 0) + i * bn
    x = jnp.where(rows == 0, 0.0, x_ref[...])
    for a in range(_A):
        ts = alpha * x + (1.0 - alpha) * pos_ref[a + 1][None]
        ts_ref[a] = jnp.where(rows == 0, 1.0 - alpha, ts)
    pre2_ref[...] = (
        jnp.dot(x, w2_ref[...], preferred_element_type=jnp.float32,
                precision=lax.Precision.HIGHEST)
        + b2_ref[...]
    )


def _prestage(x, pos_pad, w2, b2, alpha_arr):
    bn = 1000
    nb = _N // bn
    return pl.pallas_call(
        functools.partial(_pre_body, bn=bn),
        grid=(nb,),
        in_specs=[
            pl.BlockSpec((bn, _D), lambda i: (i, 0)),
            pl.BlockSpec((8, _D), lambda i: (0, 0)),
            pl.BlockSpec((_D, _D), lambda i: (0, 0)),
            pl.BlockSpec((1, _D), lambda i: (0, 0)),
            pl.BlockSpec(memory_space=pltpu.SMEM),
        ],
        out_specs=[
            pl.BlockSpec((_A, bn, _D), lambda i: (0, i, 0)),
            pl.BlockSpec((bn, _D), lambda i: (i, 0)),
        ],
        out_shape=[
            jax.ShapeDtypeStruct((_A, _N, _D), jnp.float32),
            jax.ShapeDtypeStruct((_N, _D), jnp.float32),
        ],
    )(x, pos_pad, w2, b2, alpha_arr)


_S = 6
_NSC = 26
_PER_W = _S * _NSC
_TAIL = _NCHUNK - _PER_W * _NW


def _compute_chunk(g_v, r_v):
    def _edge(e, _):
        r0 = 4 * e
        for jj in range(8):
            sl = pl.ds(16 * jj, 16)
            s0 = g_v[r0, sl]
            s1 = g_v[r0 + 1, sl]
            s2 = g_v[r0 + 2, sl]
            s3 = g_v[r0 + 3, sl]
            rr = r_v[e, sl]
            p01r = (s0 * s1) * rr
            p23r = (s2 * s3) * rr
            g_v[r0, sl] = s1 * p23r
            g_v[r0 + 1, sl] = s0 * p23r
            g_v[r0 + 2, sl] = p01r * s3
            g_v[r0 + 3, sl] = p01r * s2
        return 0
    lax.fori_loop(0, _C, _edge, 0)


def _sc_body(ts_hbm, relt_hbm, idx3_hbm, out_hbm,
             g_v0, g_v1, r_v0, r_v1, i_v0, i_v1, relt_sh, agg_sh,
             sem_ts0, sem_ts1, sem_rel0, sem_rel1, sem_idx0, sem_idx1,
             sem_sc0, sem_sc1):
    c = lax.axis_index("c")
    s = lax.axis_index("s")
    wid = s * 2 + c
    g_v = (g_v0, g_v1)
    r_v = (r_v0, r_v1)
    i_v = (i_v0, i_v1)
    sem_ts = (sem_ts0, sem_ts1)
    sem_rel = (sem_rel0, sem_rel1)
    sem_idx = (sem_idx0, sem_idx1)
    sem_sc = (sem_sc0, sem_sc1)
    @pl.when(s == 0)
    def _stage_relt():
        pltpu.sync_copy(relt_hbm, relt_sh)

    def _zrow(i, _):
        for j in range(8):
            g_v0[i, pl.ds(16 * j, 16)] = jnp.zeros((16,), jnp.float32)
        return 0
    lax.fori_loop(0, 16, _zrow, 0)
    zbase = s * _RPT
    nblk = jnp.where(s == 15, 40, 39)

    def _zcopy(i, _):
        pltpu.sync_copy(g_v0.at[pl.ds(0, 16)],
                        agg_sh.at[pl.ds(zbase + i * 16, 16)])
        return 0
    lax.fori_loop(0, nblk, _zcopy, 0)
    plsc.subcore_barrier()

    start_sc = wid * _NSC

    def _fire_gather(b, ip, j):
        pltpu.async_copy(ts_hbm.at[i_v[ip].at[j, 0]], g_v[b], sem_ts[b])
        pltpu.async_copy(relt_sh.at[i_v[ip].at[j, 2, pl.ds(0, 32)]],
                         r_v[b], sem_rel[b])

    def _wait_gather(b):
        pltpu.make_async_copy(ts_hbm.at[i_v[0].at[0, 0]], g_v[b],
                              sem_ts[b]).wait()
        pltpu.make_async_copy(relt_sh.at[i_v[0].at[0, 2, pl.ds(0, 32)]],
                              r_v[b], sem_rel[b]).wait()

    def _fire_scatter(b, ip, j):
        pltpu.async_copy(g_v[b], agg_sh.at[i_v[ip].at[j, 1]], sem_sc[b],
                         add=True)

    def _wait_scatter(b):
        pltpu.make_async_copy(g_v[b], agg_sh.at[i_v[0].at[0, 1]],
                              sem_sc[b]).wait()

    pltpu.sync_copy(idx3_hbm.at[pl.ds(start_sc * _S, _S)], i_v[0])
    _fire_gather(0, 0, 0)

    def _superpair(t, _):
        for p in range(2):
            sc = 2 * t + p
            ip = p
            def _pf():
                pltpu.async_copy(
                    idx3_hbm.at[pl.ds((start_sc + sc + 1) * _S, _S)],
                    i_v[1 - ip], sem_idx[1 - ip])
            if p == 0:
                _pf()
            else:
                pl.when(t < _NSC // 2 - 1)(_pf)
            for j in range(_S):
                b = j % 2
                _wait_gather(b)
                def _wprev():
                    _wait_scatter(1 - b)
                if p == 0 and j == 0:
                    pl.when(t > 0)(_wprev)
                else:
                    _wprev()
                if j < _S - 1:
                    _fire_gather(1 - b, ip, j + 1)
                else:
                    def _xfire():
                        pltpu.make_async_copy(
                            idx3_hbm.at[pl.ds(0, _S)], i_v[1 - ip],
                            sem_idx[1 - ip]).wait()
                        _fire_gather(1 - b, 1 - ip, 0)
                    if p == 0:
                        _xfire()
                    else:
                        pl.when(t < _NSC // 2 - 1)(_xfire)
                _compute_chunk(g_v[b], r_v[b])
                _fire_scatter(b, ip, j)
        return 0
    lax.fori_loop(0, _NSC // 2, _superpair, 0, unroll=False)
    _wait_scatter(1)

    @pl.when(wid < _TAIL)
    def _tail_chunk():
        tc = _PER_W * _NW + wid
        pltpu.sync_copy(idx3_hbm.at[pl.ds(tc, 1)], i_v[0].at[pl.ds(0, 1)])
        ts_cp = pltpu.async_copy(ts_hbm.at[i_v[0].at[0, 0]], g_v[0],
                                 sem_ts[0])
        rel_cp = pltpu.async_copy(relt_sh.at[i_v[0].at[0, 2, pl.ds(0, 32)]],
                                  r_v[0], sem_rel[0])
        ts_cp.wait()
        rel_cp.wait()
        _compute_chunk(g_v[0], r_v[0])
        pltpu.sync_copy(g_v[0], agg_sh.at[i_v[0].at[0, 1]], add=True)

    plsc.subcore_barrier()
    rbase = s * _RPT
    pltpu.sync_copy(agg_sh.at[pl.ds(rbase, _RPT)],
                    out_hbm.at[pl.ds(c * _N + rbase, _RPT)])

    @pl.when(s == 15)
    def _tail():
        pltpu.sync_copy(agg_sh.at[pl.ds(16 * _RPT, _N - 16 * _RPT)],
                        out_hbm.at[pl.ds(c * _N + 16 * _RPT, _N - 16 * _RPT)])


def _sc_stage(ts, relt, idx3):
    mesh = plsc.VectorSubcoreMesh(core_axis_name="c", subcore_axis_name="s")
    f = pl.kernel(
        _sc_body,
        out_type=jax.ShapeDtypeStruct((2 * _N, _D), jnp.float32),
        mesh=mesh,
        scratch_types=[
            pltpu.VMEM((4 * _C, _D), jnp.float32),
            pltpu.VMEM((4 * _C, _D), jnp.float32),
            pltpu.VMEM((_C, _D), jnp.float32),
            pltpu.VMEM((_C, _D), jnp.float32),
            pltpu.VMEM((_S, 3, 4 * _C), jnp.int32),
            pltpu.VMEM((_S, 3, 4 * _C), jnp.int32),
            pltpu.VMEM_SHARED((32, _D), jnp.float32),
            pltpu.VMEM_SHARED((_N, _D), jnp.float32),
            pltpu.SemaphoreType.DMA,
            pltpu.SemaphoreType.DMA,
            pltpu.SemaphoreType.DMA,
            pltpu.SemaphoreType.DMA,
            pltpu.SemaphoreType.DMA,
            pltpu.SemaphoreType.DMA,
            pltpu.SemaphoreType.DMA,
            pltpu.SemaphoreType.DMA,
        ],
    )
    return f(ts, relt, idx3)


def _post_body(p0_ref, p1_ref, pre2_ref, w1_ref, g_ref, bt_ref, out_ref, *, bn):
    i = pl.program_id(0)
    rows = lax.broadcasted_iota(jnp.int32, (bn, 1), 0) + i * bn
    agg = p0_ref[...] + p1_ref[...]
    agg = jnp.where(rows == 0, 0.0, agg)
    h = (
        jnp.dot(agg, w1_ref[...], preferred_element_type=jnp.float32,
                precision=lax.Precision.HIGHEST)
        + pre2_ref[...]
    )
    mu = jnp.mean(h, axis=-1, keepdims=True)
    dlt = h - mu
    var = jnp.mean(dlt * dlt, axis=-1, keepdims=True)
    out_ref[...] = dlt * lax.rsqrt(var + 1e-5) * g_ref[...] + bt_ref[...]


def _poststage(partials, pre2, w1, gamma2, beta2):
    bn = 1000
    nb = _N // bn
    return pl.pallas_call(
        functools.partial(_post_body, bn=bn),
        grid=(nb,),
        in_specs=[
            pl.BlockSpec((bn, _D), lambda i: (i, 0)),
            pl.BlockSpec((bn, _D), lambda i: (nb + i, 0)),
            pl.BlockSpec((bn, _D), lambda i: (i, 0)),
            pl.BlockSpec((_D, _D), lambda i: (0, 0)),
            pl.BlockSpec((1, _D), lambda i: (0, 0)),
            pl.BlockSpec((1, _D), lambda i: (0, 0)),
        ],
        out_specs=pl.BlockSpec((bn, _D), lambda i: (i, 0)),
        out_shape=jax.ShapeDtypeStruct((_N, _D), jnp.float32),
    )(partials, partials, pre2, w1, gamma2, beta2)


def kernel(node_features, query, edge_list, rel, rel_emb, pos_emb, alpha,
           W, b, gamma, beta):
    B, N, d = node_features.shape
    E, A = edge_list.shape
    x = node_features.reshape(N, d)
    pos_pad = jnp.zeros((8, d), jnp.float32).at[: A + 1].set(pos_emb)
    relt = rel_emb.at[0].set(jnp.ones((d,), rel_emb.dtype))
    w1 = W[:d]
    w2 = W[d:]
    alpha_arr = jnp.asarray(alpha, jnp.float32).reshape(1, 1)
    b2 = b.reshape(1, d)
    sidx = edge_list.astype(jnp.int32).reshape(_NCHUNK, 1, 4 * _C)
    pat = (jnp.arange(4 * _C, dtype=jnp.int32) % A) * N
    gidx = sidx + pat[None, None, :]
    relp = jnp.pad(rel.astype(jnp.int32).reshape(_NCHUNK, _C),
                   ((0, 0), (0, 3 * _C))).reshape(_NCHUNK, 1, 4 * _C)
    idx3 = jnp.concatenate([gidx, sidx, relp], axis=1)

    ts3, pre2 = _prestage(x, pos_pad, w2, b2, alpha_arr)
    ts = ts3.reshape(A * N, d)
    idx3 = pltpu.with_memory_space_constraint(idx3, pltpu.HBM)
    partials = _sc_stage(ts, relt, idx3)
    out = _poststage(partials, pre2, w1, gamma.reshape(1, d),
                     beta.reshape(1, d))
    return out.reshape(B, N, d)
